# initial kernel scaffold (unmeasured)
import jax
import jax.numpy as jnp
from jax import lax
from jax.experimental import pallas as pl
from jax.experimental.pallas import tpu as pltpu

N_DEV = 4
B, Sq, Hq, Dh = 2, 256, 4, 64
SKV_SHARD = 256
SKV = N_DEV * SKV_SHARD


def kernel(x, Wq, K_ext, V_ext, Wo):
    Kp = jnp.transpose(K_ext, (0, 2, 1, 3)).reshape(B * Hq, SKV_SHARD, Dh)
    Vp = jnp.transpose(V_ext, (0, 2, 1, 3)).reshape(B * Hq, SKV_SHARD, Dh)

    def body(x_ref, wq_ref, k_ref, v_ref, wo_ref, out_ref,
             kbuf, vbuf, k_send, k_recv, v_send, v_recv):
        my_pos = lax.axis_index("i")
        left = lax.rem(my_pos - 1 + N_DEV, N_DEV)
        right = lax.rem(my_pos + 1, N_DEV)

        barrier_sem = pltpu.get_barrier_semaphore()
        for nbr in (left, right):
            pl.semaphore_signal(
                barrier_sem, inc=1,
                device_id=(nbr,), device_id_type=pl.DeviceIdType.MESH,
            )
        pl.semaphore_wait(barrier_sem, 2)

        kbuf[0] = k_ref[...]
        vbuf[0] = v_ref[...]

        for h in range(N_DEV - 1):
            rk = pltpu.make_async_remote_copy(
                src_ref=kbuf.at[h], dst_ref=kbuf.at[h + 1],
                send_sem=k_send.at[h], recv_sem=k_recv.at[h],
                device_id=(right,), device_id_type=pl.DeviceIdType.MESH,
            )
            rv = pltpu.make_async_remote_copy(
                src_ref=vbuf.at[h], dst_ref=vbuf.at[h + 1],
                send_sem=v_send.at[h], recv_sem=v_recv.at[h],
                device_id=(right,), device_id_type=pl.DeviceIdType.MESH,
            )
            rk.start()
            rv.start()
            rk.wait()
            rv.wait()

        qi = lax.broadcasted_iota(jnp.int32, (Sq, SKV_SHARD), 0)
        kj = lax.broadcasted_iota(jnp.int32, (Sq, SKV_SHARD), 1)
        masks = []
        for s in range(N_DEV):
            origin = lax.rem(my_pos - s + N_DEV, N_DEV)
            kg = kj + origin * SKV_SHARD
            masks.append((jnp.abs(qi - kg) <= 128) | (kg < 32) | (qi < 32))

        for b in range(B):
            qb = jnp.dot(x_ref[b], wq_ref[...],
                         preferred_element_type=jnp.float32)
            ctx_heads = []
            for h in range(Hq):
                q = qb[:, h * Dh:(h + 1) * Dh]
                parts = []
                for s in range(N_DEV):
                    sc = jax.lax.dot_general(
                        q, kbuf[s, b * Hq + h],
                        (((1,), (1,)), ((), ())),
                        preferred_element_type=jnp.float32,
                    ) * 0.125
                    parts.append(jnp.where(masks[s], sc, -1e9))
                scores = jnp.concatenate(parts, axis=1)
                m = jnp.max(scores, axis=1, keepdims=True)
                w = jnp.exp(scores - m)
                w = w / jnp.sum(w, axis=1, keepdims=True)
                ctx = jnp.zeros((Sq, Dh), jnp.float32)
                for s in range(N_DEV):
                    ctx = ctx + jnp.dot(
                        w[:, s * SKV_SHARD:(s + 1) * SKV_SHARD],
                        vbuf[s, b * Hq + h],
                        preferred_element_type=jnp.float32,
                    )
                ctx_heads.append(ctx)
            ctx_b = jnp.concatenate(ctx_heads, axis=1)
            out_ref[b] = jnp.dot(ctx_b, wo_ref[...],
                                 preferred_element_type=jnp.float32)

    return pl.pallas_call(
        body,
        out_shape=jax.ShapeDtypeStruct((B, Sq, 512), jnp.float32),
        in_specs=[pl.BlockSpec(memory_space=pltpu.VMEM)] * 5,
        out_specs=pl.BlockSpec(memory_space=pltpu.VMEM),
        scratch_shapes=[
            pltpu.VMEM((N_DEV, B * Hq, SKV_SHARD, Dh), jnp.float32),
            pltpu.VMEM((N_DEV, B * Hq, SKV_SHARD, Dh), jnp.float32),
            pltpu.SemaphoreType.DMA((N_DEV - 1,)),
            pltpu.SemaphoreType.DMA((N_DEV - 1,)),
            pltpu.SemaphoreType.DMA((N_DEV - 1,)),
            pltpu.SemaphoreType.DMA((N_DEV - 1,)),
        ],
        compiler_params=pltpu.CompilerParams(collective_id=0),
    )(x, Wq, Kp, Vp)


# baseline (device time: 85157 ns/iter reference)
import jax
import jax.numpy as jnp
from jax import lax
from jax.experimental import pallas as pl
from jax.experimental.pallas import tpu as pltpu

N_DEV = 4
B, Sq, Hq, Dh = 2, 256, 4, 64
SKV_SHARD = 256
SKV = N_DEV * SKV_SHARD


def kernel(x, Wq, K_ext, V_ext, Wo):
    Kp = jnp.transpose(K_ext, (0, 2, 1, 3)).reshape(B * Hq, SKV_SHARD, Dh)
    Vp = jnp.transpose(V_ext, (0, 2, 1, 3)).reshape(B * Hq, SKV_SHARD, Dh)

    def body(x_ref, wq_ref, k_ref, v_ref, wo_ref, out_ref,
             kbuf, vbuf, k_send, k_recv, v_send, v_recv):
        my_pos = lax.axis_index("i")
        left = lax.rem(my_pos - 1 + N_DEV, N_DEV)
        right = lax.rem(my_pos + 1, N_DEV)

        barrier_sem = pltpu.get_barrier_semaphore()
        for nbr in (left, right):
            pl.semaphore_signal(
                barrier_sem, inc=1,
                device_id=(nbr,), device_id_type=pl.DeviceIdType.MESH,
            )
        pl.semaphore_wait(barrier_sem, 2)

        kbuf[0] = k_ref[...]
        vbuf[0] = v_ref[...]

        for h in range(N_DEV - 1):
            rk = pltpu.make_async_remote_copy(
                src_ref=kbuf.at[h], dst_ref=kbuf.at[h + 1],
                send_sem=k_send.at[h], recv_sem=k_recv.at[h],
                device_id=(right,), device_id_type=pl.DeviceIdType.MESH,
            )
            rv = pltpu.make_async_remote_copy(
                src_ref=vbuf.at[h], dst_ref=vbuf.at[h + 1],
                send_sem=v_send.at[h], recv_sem=v_recv.at[h],
                device_id=(right,), device_id_type=pl.DeviceIdType.MESH,
            )
            rk.start()
            rv.start()
            rk.wait()
            rv.wait()

        qi = lax.broadcasted_iota(jnp.int32, (Sq, SKV_SHARD), 0)
        kj = lax.broadcasted_iota(jnp.int32, (Sq, SKV_SHARD), 1)
        masks = []
        for s in range(N_DEV):
            origin = lax.rem(my_pos - s + N_DEV, N_DEV)
            kg = kj + origin * SKV_SHARD
            masks.append((jnp.abs(qi - kg) <= 128) | (kg < 32) | (qi < 32))

        for b in range(B):
            qb = jnp.dot(x_ref[b], wq_ref[...],
                         preferred_element_type=jnp.float32)
            ctx_heads = []
            for h in range(Hq):
                q = qb[:, h * Dh:(h + 1) * Dh]
                parts = []
                for s in range(N_DEV):
                    sc = jax.lax.dot_general(
                        q, kbuf[s, b * Hq + h],
                        (((1,), (1,)), ((), ())),
                        preferred_element_type=jnp.float32,
                    ) * 0.125
                    parts.append(jnp.where(masks[s], sc, -1e9))
                scores = jnp.concatenate(parts, axis=1)
                m = jnp.max(scores, axis=1, keepdims=True)
                w = jnp.exp(scores - m)
                w = w / jnp.sum(w, axis=1, keepdims=True)
                ctx = jnp.zeros((Sq, Dh), jnp.float32)
                for s in range(N_DEV):
                    ctx = ctx + jnp.dot(
                        w[:, s * SKV_SHARD:(s + 1) * SKV_SHARD],
                        vbuf[s, b * Hq + h],
                        preferred_element_type=jnp.float32,
                    )
                ctx_heads.append(ctx)
            ctx_b = jnp.concatenate(ctx_heads, axis=1)
            out_ref[b] = jnp.dot(ctx_b, wo_ref[...],
                                 preferred_element_type=jnp.float32)

    return pl.pallas_call(
        body,
        out_shape=jax.ShapeDtypeStruct((B, Sq, 512), jnp.float32),
        in_specs=[pl.BlockSpec(memory_space=pltpu.VMEM)] * 5,
        out_specs=pl.BlockSpec(memory_space=pltpu.VMEM),
        scratch_shapes=[
            pltpu.VMEM((N_DEV, B * Hq, SKV_SHARD, Dh), jnp.float32),
            pltpu.VMEM((N_DEV, B * Hq, SKV_SHARD, Dh), jnp.float32),
            pltpu.SemaphoreType.DMA((N_DEV - 1,)),
            pltpu.SemaphoreType.DMA((N_DEV - 1,)),
            pltpu.SemaphoreType.DMA((N_DEV - 1,)),
            pltpu.SemaphoreType.DMA((N_DEV - 1,)),
        ],
        compiler_params=pltpu.CompilerParams(collective_id=0),
    )(x, Wq, Kp, Vp, Wo)


# device time: 17363 ns/iter; 4.9045x vs baseline; 4.9045x over previous
import jax
import jax.numpy as jnp
from jax import lax
from jax.experimental import pallas as pl
from jax.experimental.pallas import tpu as pltpu

N_DEV = 4
B, Sq, Hq, Dh = 2, 256, 4, 64
SKV_SHARD = 256
D = Hq * Dh


def kernel(x, Wq, K_ext, V_ext, Wo):
    Kp = jnp.transpose(K_ext, (0, 2, 1, 3)).reshape(B * Hq, SKV_SHARD, Dh)
    Vp = jnp.transpose(V_ext, (0, 2, 1, 3)).reshape(B * Hq, SKV_SHARD, Dh)

    def body(x_ref, wq_ref, k_ref, v_ref, wo_ref, out_ref,
             ctx_sbuf, den_sbuf, ctx_rbuf, den_rbuf,
             ctx_send, ctx_recv, den_send, den_recv):
        my_pos = lax.axis_index("i")
        peers = [
            lax.rem(my_pos + 1, N_DEV),
            lax.rem(my_pos - 1 + N_DEV, N_DEV),
            lax.rem(my_pos + 2, N_DEV),
        ]

        barrier_sem = pltpu.get_barrier_semaphore()
        for p in peers:
            pl.semaphore_signal(
                barrier_sem, inc=1,
                device_id=(p,), device_id_type=pl.DeviceIdType.MESH,
            )

        qi = lax.broadcasted_iota(jnp.int32, (Sq, SKV_SHARD), 0)
        kg = lax.broadcasted_iota(jnp.int32, (Sq, SKV_SHARD), 1) \
            + my_pos * SKV_SHARD
        mask = (jnp.abs(qi - kg) <= 128) | (kg < 32) | (qi < 32)

        for b in range(B):
            qb = jnp.dot(x_ref[b], wq_ref[...],
                         preferred_element_type=jnp.float32)
            for h in range(Hq):
                q = qb[:, h * Dh:(h + 1) * Dh]
                sc = jax.lax.dot_general(
                    q, k_ref[b * Hq + h],
                    (((1,), (1,)), ((), ())),
                    preferred_element_type=jnp.float32,
                ) * 0.125
                e = jnp.where(mask, jnp.exp(sc), 0.0)
                den_sbuf[b * Hq + h] = jnp.sum(e, axis=1).astype(jnp.bfloat16)
                ctx_sbuf[b * Sq:(b + 1) * Sq, h * Dh:(h + 1) * Dh] = jnp.dot(
                    e, v_ref[b * Hq + h],
                    preferred_element_type=jnp.float32,
                ).astype(jnp.bfloat16)

        pl.semaphore_wait(barrier_sem, N_DEV - 1)

        rdmas = []
        for t, p in enumerate(peers):
            rc = pltpu.make_async_remote_copy(
                src_ref=ctx_sbuf, dst_ref=ctx_rbuf.at[t],
                send_sem=ctx_send.at[t], recv_sem=ctx_recv.at[t],
                device_id=(p,), device_id_type=pl.DeviceIdType.MESH,
            )
            rd = pltpu.make_async_remote_copy(
                src_ref=den_sbuf, dst_ref=den_rbuf.at[t],
                send_sem=den_send.at[t], recv_sem=den_recv.at[t],
                device_id=(p,), device_id_type=pl.DeviceIdType.MESH,
            )
            rc.start()
            rd.start()
            rdmas.append((rc, rd))
        for rc, rd in rdmas:
            rc.wait()
            rd.wait()

        for b in range(B):
            ctx = ctx_sbuf[b * Sq:(b + 1) * Sq, :].astype(jnp.float32)
            den = den_sbuf[b * Hq:(b + 1) * Hq, :].astype(jnp.float32)
            for t in range(N_DEV - 1):
                ctx = ctx + ctx_rbuf[t, b * Sq:(b + 1) * Sq, :].astype(
                    jnp.float32)
                den = den + den_rbuf[t, b * Hq:(b + 1) * Hq, :].astype(
                    jnp.float32)
            div = jnp.broadcast_to(
                jnp.transpose(den)[:, :, None], (Sq, Hq, Dh)
            ).reshape(Sq, D)
            out_ref[b] = jnp.dot(ctx / div, wo_ref[...],
                                 preferred_element_type=jnp.float32)

    return pl.pallas_call(
        body,
        out_shape=jax.ShapeDtypeStruct((B, Sq, 512), jnp.float32),
        in_specs=[pl.BlockSpec(memory_space=pltpu.VMEM)] * 5,
        out_specs=pl.BlockSpec(memory_space=pltpu.VMEM),
        scratch_shapes=[
            pltpu.VMEM((B * Sq, D), jnp.bfloat16),
            pltpu.VMEM((B * Hq, Sq), jnp.bfloat16),
            pltpu.VMEM((N_DEV - 1, B * Sq, D), jnp.bfloat16),
            pltpu.VMEM((N_DEV - 1, B * Hq, Sq), jnp.bfloat16),
            pltpu.SemaphoreType.DMA((N_DEV - 1,)),
            pltpu.SemaphoreType.DMA((N_DEV - 1,)),
            pltpu.SemaphoreType.DMA((N_DEV - 1,)),
            pltpu.SemaphoreType.DMA((N_DEV - 1,)),
        ],
        compiler_params=pltpu.CompilerParams(collective_id=0),
    )(x, Wq, Kp, Vp, Wo)


# device time: 16174 ns/iter; 5.2651x vs baseline; 1.0735x over previous
import jax
import jax.numpy as jnp
from jax import lax
from jax.experimental import pallas as pl
from jax.experimental.pallas import tpu as pltpu

N_DEV = 4
B, Sq, Hq, Dh = 2, 256, 4, 64
SKV_SHARD = 256
D = Hq * Dh
R = Sq + Hq


def kernel(x, Wq, K_ext, V_ext, Wo):
    Kp = jnp.transpose(K_ext, (0, 2, 1, 3)).reshape(B * Hq, SKV_SHARD, Dh)
    Vp = jnp.transpose(V_ext, (0, 2, 1, 3)).reshape(B * Hq, SKV_SHARD, Dh)

    def body(x_ref, wq_ref, k_ref, v_ref, wo_ref, out_ref,
             sbuf, rbuf, send_sems, recv_sems):
        my_pos = lax.axis_index("i")
        peers = [
            lax.rem(my_pos + 1, N_DEV),
            lax.rem(my_pos - 1 + N_DEV, N_DEV),
            lax.rem(my_pos + 2, N_DEV),
        ]

        barrier_sem = pltpu.get_barrier_semaphore()
        for p in peers:
            pl.semaphore_signal(
                barrier_sem, inc=1,
                device_id=(p,), device_id_type=pl.DeviceIdType.MESH,
            )

        qi = lax.broadcasted_iota(jnp.int32, (Sq, SKV_SHARD), 0)
        kg = lax.broadcasted_iota(jnp.int32, (Sq, SKV_SHARD), 1) \
            + my_pos * SKV_SHARD
        mask = (jnp.abs(qi - kg) <= 128) | (kg < 32) | (qi < 32)

        rdmas = {}
        for b in range(B):
            qb = jnp.dot(x_ref[b], wq_ref[...],
                         preferred_element_type=jnp.float32)
            for h in range(Hq):
                q = qb[:, h * Dh:(h + 1) * Dh]
                sc = jax.lax.dot_general(
                    q, k_ref[b * Hq + h],
                    (((1,), (1,)), ((), ())),
                    preferred_element_type=jnp.float32,
                ) * 0.125
                e = jnp.where(mask, jnp.exp(sc), 0.0)
                sbuf[b, Sq + h] = jnp.sum(e, axis=1).astype(jnp.bfloat16)
                sbuf[b, 0:Sq, h * Dh:(h + 1) * Dh] = jnp.dot(
                    e, v_ref[b * Hq + h],
                    preferred_element_type=jnp.float32,
                ).astype(jnp.bfloat16)
            if b == 0:
                pl.semaphore_wait(barrier_sem, N_DEV - 1)
            for t, p in enumerate(peers):
                r = pltpu.make_async_remote_copy(
                    src_ref=sbuf.at[b], dst_ref=rbuf.at[t, b],
                    send_sem=send_sems.at[b, t], recv_sem=recv_sems.at[b, t],
                    device_id=(p,), device_id_type=pl.DeviceIdType.MESH,
                )
                r.start()
                rdmas[(b, t)] = r

        for b in range(B):
            for t in range(N_DEV - 1):
                rdmas[(b, t)].wait_recv()
            ctx = sbuf[b, 0:Sq, :].astype(jnp.float32)
            den = sbuf[b, Sq:R, :].astype(jnp.float32)
            for t in range(N_DEV - 1):
                ctx = ctx + rbuf[t, b, 0:Sq, :].astype(jnp.float32)
                den = den + rbuf[t, b, Sq:R, :].astype(jnp.float32)
            div = jnp.broadcast_to(
                jnp.transpose(den)[:, :, None], (Sq, Hq, Dh)
            ).reshape(Sq, D)
            out_ref[b] = jnp.dot(ctx / div, wo_ref[...],
                                 preferred_element_type=jnp.float32)

        for r in rdmas.values():
            r.wait_send()

    return pl.pallas_call(
        body,
        out_shape=jax.ShapeDtypeStruct((B, Sq, 512), jnp.float32),
        in_specs=[pl.BlockSpec(memory_space=pltpu.VMEM)] * 5,
        out_specs=pl.BlockSpec(memory_space=pltpu.VMEM),
        scratch_shapes=[
            pltpu.VMEM((B, R, D), jnp.bfloat16),
            pltpu.VMEM((N_DEV - 1, B, R, D), jnp.bfloat16),
            pltpu.SemaphoreType.DMA((B, N_DEV - 1)),
            pltpu.SemaphoreType.DMA((B, N_DEV - 1)),
        ],
        compiler_params=pltpu.CompilerParams(collective_id=0),
    )(x, Wq, Kp, Vp, Wo)


# device time: 13205 ns/iter; 6.4488x vs baseline; 1.2248x over previous
import jax
import jax.numpy as jnp
from jax import lax
from jax.experimental import pallas as pl
from jax.experimental.pallas import tpu as pltpu

N_DEV = 4
B, Sq, Hq, Dh = 2, 256, 4, 64
SKV_SHARD = 256
D = Hq * Dh
R = Sq + Hq

_RANGES = [
    [(0, R)],
    [(0, 32), (128, R - 128)],
    [(0, 32), (Sq, Hq)],
    [(0, 32), (Sq, Hq)],
]


def kernel(x, Wq, K_ext, V_ext, Wo):
    Kp = jnp.transpose(K_ext, (0, 2, 1, 3)).reshape(B * Hq, SKV_SHARD, Dh)
    Vp = jnp.transpose(V_ext, (0, 2, 1, 3)).reshape(B * Hq, SKV_SHARD, Dh)

    def body(x_ref, wq_ref, k_ref, v_ref, wo_ref, out_ref,
             sbuf, rbuf, send_sems, recv_sems):
        my_pos = lax.axis_index("i")
        peers = [
            lax.rem(my_pos + 1, N_DEV),
            lax.rem(my_pos - 1 + N_DEV, N_DEV),
            lax.rem(my_pos + 2, N_DEV),
        ]
        origin_of = lambda c, t: [(c - 1) % N_DEV, (c + 1) % N_DEV,
                                  (c + 2) % N_DEV][t]

        rbuf[...] = jnp.zeros((N_DEV - 1, B, R, D), jnp.bfloat16)

        barrier_sem = pltpu.get_barrier_semaphore()
        for p in peers:
            pl.semaphore_signal(
                barrier_sem, inc=1,
                device_id=(p,), device_id_type=pl.DeviceIdType.MESH,
            )

        qi = lax.broadcasted_iota(jnp.int32, (Sq, SKV_SHARD), 0)
        kg = lax.broadcasted_iota(jnp.int32, (Sq, SKV_SHARD), 1) \
            + my_pos * SKV_SHARD
        mask = (jnp.abs(qi - kg) <= 128) | (kg < 32) | (qi < 32)

        def descriptor(b, t, p, j, r0, n):
            return pltpu.make_async_remote_copy(
                src_ref=sbuf.at[b, pl.ds(r0, n)],
                dst_ref=rbuf.at[t, b, pl.ds(r0, n)],
                send_sem=send_sems.at[b, t, j],
                recv_sem=recv_sems.at[b, t, j],
                device_id=(p,), device_id_type=pl.DeviceIdType.MESH,
            )

        for b in range(B):
            qb = jnp.dot(x_ref[b], wq_ref[...],
                         preferred_element_type=jnp.float32)
            for h in range(Hq):
                q = qb[:, h * Dh:(h + 1) * Dh]
                sc = jax.lax.dot_general(
                    q, k_ref[b * Hq + h],
                    (((1,), (1,)), ((), ())),
                    preferred_element_type=jnp.float32,
                ) * 0.125
                e = jnp.where(mask, jnp.exp(sc), 0.0)
                sbuf[b, Sq + h] = jnp.sum(e, axis=1).astype(jnp.bfloat16)
                sbuf[b, 0:Sq, h * Dh:(h + 1) * Dh] = jnp.dot(
                    e, v_ref[b * Hq + h],
                    preferred_element_type=jnp.float32,
                ).astype(jnp.bfloat16)
            if b == 0:
                pl.semaphore_wait(barrier_sem, N_DEV - 1)
            for c in range(N_DEV):
                @pl.when(my_pos == c)
                def _(b=b, c=c):
                    for t, p in enumerate(peers):
                        for j, (r0, n) in enumerate(_RANGES[c]):
                            descriptor(b, t, p, j, r0, n).start()

        for b in range(B):
            for c in range(N_DEV):
                @pl.when(my_pos == c)
                def _(b=b, c=c):
                    for t, p in enumerate(peers):
                        for j, (r0, n) in enumerate(_RANGES[origin_of(c, t)]):
                            descriptor(b, t, p, j, r0, n).wait_recv()
            ctx = sbuf[b, 0:Sq, :].astype(jnp.float32)
            den = sbuf[b, Sq:R, :].astype(jnp.float32)
            for t in range(N_DEV - 1):
                ctx = ctx + rbuf[t, b, 0:Sq, :].astype(jnp.float32)
                den = den + rbuf[t, b, Sq:R, :].astype(jnp.float32)
            div = jnp.broadcast_to(
                jnp.transpose(den)[:, :, None], (Sq, Hq, Dh)
            ).reshape(Sq, D)
            out_ref[b] = jnp.dot(ctx / div, wo_ref[...],
                                 preferred_element_type=jnp.float32)

        for c in range(N_DEV):
            @pl.when(my_pos == c)
            def _(c=c):
                for b in range(B):
                    for t, p in enumerate(peers):
                        for j, (r0, n) in enumerate(_RANGES[c]):
                            descriptor(b, t, p, j, r0, n).wait_send()

    return pl.pallas_call(
        body,
        out_shape=jax.ShapeDtypeStruct((B, Sq, 512), jnp.float32),
        in_specs=[pl.BlockSpec(memory_space=pltpu.VMEM)] * 5,
        out_specs=pl.BlockSpec(memory_space=pltpu.VMEM),
        scratch_shapes=[
            pltpu.VMEM((B, R, D), jnp.bfloat16),
            pltpu.VMEM((N_DEV - 1, B, R, D), jnp.bfloat16),
            pltpu.SemaphoreType.DMA((B, N_DEV - 1, 2)),
            pltpu.SemaphoreType.DMA((B, N_DEV - 1, 2)),
        ],
        compiler_params=pltpu.CompilerParams(collective_id=0),
    )(x, Wq, Kp, Vp, Wo)
